# baseline (speedup 1.0000x reference)
import functools

import jax
import jax.numpy as jnp
from jax import lax
from jax.experimental import pallas as pl
from jax.experimental.pallas import tpu as pltpu
from jax.experimental.pallas import tpu_sc as plsc

_BATCH = 1024
_MEM = 200
_DIM = 128
_LANES = 16
_NW = 32
_B_SC = 448
_RPW = _B_SC // _NW
_MG = (_MEM + _LANES - 1) // _LANES
_ROW = _MEM * _DIM
_CH = _DIM // _LANES
_OUT_W = _RPW * _MEM
_NBUF = 2
_BB = 64

_GATHER_DNUMS = lax.GatherDimensionNumbers(
    offset_dims=(), collapsed_slice_dims=(0,), start_index_map=(0,))


def _permute(v, idx):
    return lax.gather(v, idx[:, None], _GATHER_DNUMS, (1,),
                      mode=lax.GatherScatterMode.PROMISE_IN_BOUNDS)


def _sc_body(q_hbm, mem_hbm, soft_hbm, logit_hbm,
             q_v, mem_v, soft_v, logit_v, sem_q, sem_m0, sem_m1, sem_o):
    cid = lax.axis_index("c")
    sid = lax.axis_index("s")
    wid = sid * 2 + cid
    base = wid * _RPW

    lane = lax.iota(jnp.int32, _LANES)
    tail_n = _MEM - (_MG - 1) * _LANES
    tail_mask = lane < tail_n
    lane_eq = [lane == j for j in range(_LANES)]
    fold_idx = [jnp.bitwise_xor(lane, w) for w in (8, 4, 2, 1)]
    neg_inf = jnp.full((_LANES,), -3.0e38, jnp.float32)
    zero16 = jnp.zeros((_LANES,), jnp.float32)

    sems = (sem_m0, sem_m1)

    def start_row(b, buf):
        pltpu.async_copy(mem_hbm.at[pl.ds((base + b) * _ROW, _ROW)],
                         mem_v.at[pl.ds(buf * _ROW, _ROW)], sems[buf])

    def wait_row(b, buf):
        pltpu.make_async_copy(mem_hbm.at[pl.ds((base + b) * _ROW, _ROW)],
                              mem_v.at[pl.ds(buf * _ROW, _ROW)],
                              sems[buf]).wait()

    pltpu.async_copy(q_hbm.at[pl.ds(base * _DIM, _RPW * _DIM)], q_v, sem_q)
    for buf in range(_NBUF):
        start_row(buf, buf)
    pltpu.make_async_copy(q_hbm.at[pl.ds(base * _DIM, _RPW * _DIM)],
                          q_v, sem_q).wait()

    def crosslane_sum(v):
        for idx in fold_idx:
            v = v + _permute(v, idx)
        return v

    def dot_row(bl, buf):
        qbase = bl * _DIM
        qc = [q_v[pl.ds(qbase + k * _LANES, _LANES)] for k in range(_CH)]
        vbase = buf * _ROW
        row0 = bl * _MEM

        def mg_body(mg, carry):
            goff = vbase + mg * (_LANES * _DIM)
            grp = zero16
            for j in range(_LANES):
                joff = goff + j * _DIM
                acc = mem_v[pl.ds(joff, _LANES)] * qc[0]
                for k in range(1, _CH):
                    acc = acc + mem_v[pl.ds(joff + k * _LANES, _LANES)] * qc[k]
                grp = jnp.where(lane_eq[j], crosslane_sum(acc), grp)
            logit_v[pl.ds(row0 + mg * _LANES, _LANES)] = grp
            return carry

        lax.fori_loop(0, _MG, mg_body, 0)

    def softmax_row(bl):
        row0 = bl * _MEM
        accs = [logit_v[pl.ds(row0 + mg * _LANES, _LANES)]
                for mg in range(_MG)]
        masked_last = jnp.where(tail_mask, accs[_MG - 1], neg_inf)
        vmax = masked_last
        for mg in range(_MG - 1):
            vmax = jnp.maximum(vmax, accs[mg])
        red = vmax
        for idx in fold_idx:
            red = jnp.maximum(red, _permute(red, idx))
        exps = [jnp.exp(a - red) for a in accs[:-1]]
        exps.append(jnp.where(tail_mask, jnp.exp(masked_last - red), zero16))
        vsum = exps[0]
        for e in exps[1:]:
            vsum = vsum + e
        tot = crosslane_sum(vsum)
        inv = jnp.full((_LANES,), 1.0, jnp.float32) / tot
        for mg in range(_MG):
            soft_v[pl.ds(row0 + mg * _LANES, _LANES)] = exps[mg] * inv

    def rows_body(i, carry):
        for buf in range(_NBUF):
            b = _NBUF * i + buf
            wait_row(b, buf)
            dot_row(b, buf)

            @pl.when(i < _RPW // _NBUF - 1)
            def _():
                start_row(b + _NBUF, buf)

            softmax_row(b)
        return carry

    lax.fori_loop(0, _RPW // _NBUF, rows_body, 0)

    out0 = wid * _OUT_W
    pltpu.async_copy(soft_v.at[pl.ds(0, _OUT_W)],
                     soft_hbm.at[pl.ds(out0, _OUT_W)], sem_o)
    pltpu.make_async_copy(soft_v.at[pl.ds(0, _OUT_W)],
                          soft_hbm.at[pl.ds(out0, _OUT_W)], sem_o).wait()
    pltpu.sync_copy(logit_v.at[pl.ds(0, _OUT_W)],
                    logit_hbm.at[pl.ds(out0, _OUT_W)])


def _tc_body(q_ref, m_ref, soft_ref, logit_ref, lt_ref):
    qt = q_ref[...].T
    for b in range(_BB):
        lt_ref[:, b : b + 1] = jax.lax.dot(
            m_ref[b], qt[:, b : b + 1], preferred_element_type=jnp.float32
        )
    lt = lt_ref[...]
    logit_ref[...] = lt.T
    mx = jnp.max(lt, axis=0, keepdims=True)
    e = jnp.exp(lt - mx)
    soft_ref[...] = (e / jnp.sum(e, axis=0, keepdims=True)).T


@jax.jit
def kernel(query_vector, graph_out_features):
    mesh = plsc.VectorSubcoreMesh(core_axis_name="c", subcore_axis_name="s")
    sc_flat = _B_SC * _MEM
    sc_call = functools.partial(
        pl.kernel,
        mesh=mesh,
        compiler_params=pltpu.CompilerParams(needs_layout_passes=False),
        out_type=[
            jax.ShapeDtypeStruct((sc_flat,), jnp.float32),
            jax.ShapeDtypeStruct((sc_flat,), jnp.float32),
        ],
        scratch_types=[
            pltpu.VMEM((_RPW * _DIM,), jnp.float32),
            pltpu.VMEM((_NBUF * _ROW + 1024,), jnp.float32),
            pltpu.VMEM((_OUT_W + 8,), jnp.float32),
            pltpu.VMEM((_OUT_W + 8,), jnp.float32),
            pltpu.SemaphoreType.DMA,
            pltpu.SemaphoreType.DMA,
            pltpu.SemaphoreType.DMA,
            pltpu.SemaphoreType.DMA,
        ],
    )(_sc_body)

    n_tc = _BATCH - _B_SC
    off = _B_SC // _BB
    soft_tc, logit_tc = pl.pallas_call(
        _tc_body,
        grid=(n_tc // _BB,),
        in_specs=[
            pl.BlockSpec((_BB, _DIM), lambda i: (i + off, 0)),
            pl.BlockSpec((_BB, _MEM, _DIM), lambda i: (i + off, 0, 0)),
        ],
        out_specs=[
            pl.BlockSpec((_BB, _MEM), lambda i: (i, 0)),
            pl.BlockSpec((_BB, _MEM), lambda i: (i, 0)),
        ],
        out_shape=[
            jax.ShapeDtypeStruct((n_tc, _MEM), jnp.float32),
            jax.ShapeDtypeStruct((n_tc, _MEM), jnp.float32),
        ],
        scratch_shapes=[pltpu.VMEM((_MEM, _BB), jnp.float32)],
    )(query_vector, graph_out_features)

    soft_sc, logit_sc = sc_call(jnp.reshape(query_vector, (-1,)),
                                jnp.reshape(graph_out_features, (-1,)))

    soft = jnp.concatenate(
        [jnp.reshape(soft_sc, (_B_SC, _MEM)), soft_tc], axis=0)
    logits = jnp.concatenate(
        [jnp.reshape(logit_sc, (_B_SC, _MEM)), logit_tc], axis=0)
    return (soft, logits)


# speedup vs baseline: 1.0020x; 1.0020x over previous
import functools

import jax
import jax.numpy as jnp
from jax import lax
from jax.experimental import pallas as pl
from jax.experimental.pallas import tpu as pltpu
from jax.experimental.pallas import tpu_sc as plsc

_BATCH = 1024
_MEM = 200
_DIM = 128
_LANES = 16
_NW = 32
_B_SC = 448
_RPW = _B_SC // _NW
_MG = (_MEM + _LANES - 1) // _LANES
_ROW = _MEM * _DIM
_CH = _DIM // _LANES
_OUT_W = _RPW * _MEM
_NBUF = 2
_BB = 64

_GATHER_DNUMS = lax.GatherDimensionNumbers(
    offset_dims=(), collapsed_slice_dims=(0,), start_index_map=(0,))


def _permute(v, idx):
    return lax.gather(v, idx[:, None], _GATHER_DNUMS, (1,),
                      mode=lax.GatherScatterMode.PROMISE_IN_BOUNDS)


def _sc_body(q_hbm, mem_hbm, soft_hbm, logit_hbm,
             q_v, mem_v, soft_v, logit_v, sem_q, sem_m0, sem_m1, sem_o):
    cid = lax.axis_index("c")
    sid = lax.axis_index("s")
    wid = sid * 2 + cid
    base = wid * _RPW

    lane = lax.iota(jnp.int32, _LANES)
    tail_n = _MEM - (_MG - 1) * _LANES
    tail_mask = lane < tail_n
    lane_eq = [lane == j for j in range(_LANES)]
    fold_idx = [jnp.bitwise_xor(lane, w) for w in (8, 4, 2, 1)]
    neg_inf = jnp.full((_LANES,), -3.0e38, jnp.float32)
    zero16 = jnp.zeros((_LANES,), jnp.float32)

    sems = (sem_m0, sem_m1)

    def start_row(b, buf):
        pltpu.async_copy(mem_hbm.at[pl.ds((base + b) * _ROW, _ROW)],
                         mem_v.at[pl.ds(buf * _ROW, _ROW)], sems[buf])

    def wait_row(b, buf):
        pltpu.make_async_copy(mem_hbm.at[pl.ds((base + b) * _ROW, _ROW)],
                              mem_v.at[pl.ds(buf * _ROW, _ROW)],
                              sems[buf]).wait()

    pltpu.async_copy(q_hbm.at[pl.ds(base * _DIM, _RPW * _DIM)], q_v, sem_q)
    for buf in range(_NBUF):
        start_row(buf, buf)
    pltpu.make_async_copy(q_hbm.at[pl.ds(base * _DIM, _RPW * _DIM)],
                          q_v, sem_q).wait()

    def crosslane_sum(v):
        for idx in fold_idx:
            v = v + _permute(v, idx)
        return v

    def dot_row(bl, buf):
        qbase = bl * _DIM
        qc = [q_v[pl.ds(qbase + k * _LANES, _LANES)] for k in range(_CH)]
        vbase = buf * _ROW
        row0 = bl * _MEM

        def mg_body(mg, carry):
            goff = vbase + mg * (_LANES * _DIM)
            grp = zero16
            for j in range(_LANES):
                joff = goff + j * _DIM
                acc = mem_v[pl.ds(joff, _LANES)] * qc[0]
                for k in range(1, _CH):
                    acc = acc + mem_v[pl.ds(joff + k * _LANES, _LANES)] * qc[k]
                grp = jnp.where(lane_eq[j], crosslane_sum(acc), grp)
            logit_v[pl.ds(row0 + mg * _LANES, _LANES)] = grp
            return carry

        lax.fori_loop(0, _MG, mg_body, 0)

    def softmax_row(bl):
        row0 = bl * _MEM
        accs = [logit_v[pl.ds(row0 + mg * _LANES, _LANES)]
                for mg in range(_MG)]
        masked_last = jnp.where(tail_mask, accs[_MG - 1], neg_inf)
        vmax = masked_last
        for mg in range(_MG - 1):
            vmax = jnp.maximum(vmax, accs[mg])
        red = vmax
        for idx in fold_idx:
            red = jnp.maximum(red, _permute(red, idx))
        exps = [jnp.exp(a - red) for a in accs[:-1]]
        exps.append(jnp.where(tail_mask, jnp.exp(masked_last - red), zero16))
        vsum = exps[0]
        for e in exps[1:]:
            vsum = vsum + e
        tot = crosslane_sum(vsum)
        inv = jnp.full((_LANES,), 1.0, jnp.float32) / tot
        for mg in range(_MG):
            soft_v[pl.ds(row0 + mg * _LANES, _LANES)] = exps[mg] * inv

    def rows_body(i, carry):
        for buf in range(_NBUF):
            b = _NBUF * i + buf
            wait_row(b, buf)
            dot_row(b, buf)

            @pl.when(i < _RPW // _NBUF - 1)
            def _():
                start_row(b + _NBUF, buf)

            softmax_row(b)
        return carry

    lax.fori_loop(0, _RPW // _NBUF, rows_body, 0)

    out0 = wid * _OUT_W
    pltpu.async_copy(soft_v.at[pl.ds(0, _OUT_W)],
                     soft_hbm.at[pl.ds(out0, _OUT_W)], sem_o)
    pltpu.make_async_copy(soft_v.at[pl.ds(0, _OUT_W)],
                          soft_hbm.at[pl.ds(out0, _OUT_W)], sem_o).wait()
    pltpu.sync_copy(logit_v.at[pl.ds(0, _OUT_W)],
                    logit_hbm.at[pl.ds(out0, _OUT_W)])


def _tc_body(q_ref, m_ref, soft_ref, logit_ref, lt_ref):
    qt = q_ref[...].T
    for b in range(_BB):
        lt_ref[:, b : b + 1] = jax.lax.dot(
            m_ref[b], qt[:, b : b + 1], preferred_element_type=jnp.float32
        )
    lt = lt_ref[...]
    logit_ref[...] = lt.T
    mx = jnp.max(lt, axis=0, keepdims=True)
    e = jnp.exp(lt - mx)
    soft_ref[...] = (e / jnp.sum(e, axis=0, keepdims=True)).T


@jax.jit
def kernel(query_vector, graph_out_features):
    mesh = plsc.VectorSubcoreMesh(core_axis_name="c", subcore_axis_name="s")
    sc_flat = _B_SC * _MEM
    sc_call = functools.partial(
        pl.kernel,
        mesh=mesh,
        compiler_params=pltpu.CompilerParams(needs_layout_passes=False,
                                             skip_device_barrier=True),
        out_type=[
            jax.ShapeDtypeStruct((sc_flat,), jnp.float32),
            jax.ShapeDtypeStruct((sc_flat,), jnp.float32),
        ],
        scratch_types=[
            pltpu.VMEM((_RPW * _DIM,), jnp.float32),
            pltpu.VMEM((_NBUF * _ROW + 1024,), jnp.float32),
            pltpu.VMEM((_OUT_W + 8,), jnp.float32),
            pltpu.VMEM((_OUT_W + 8,), jnp.float32),
            pltpu.SemaphoreType.DMA,
            pltpu.SemaphoreType.DMA,
            pltpu.SemaphoreType.DMA,
            pltpu.SemaphoreType.DMA,
        ],
    )(_sc_body)

    n_tc = _BATCH - _B_SC
    off = _B_SC // _BB
    soft_tc, logit_tc = pl.pallas_call(
        _tc_body,
        grid=(n_tc // _BB,),
        in_specs=[
            pl.BlockSpec((_BB, _DIM), lambda i: (i + off, 0)),
            pl.BlockSpec((_BB, _MEM, _DIM), lambda i: (i + off, 0, 0)),
        ],
        out_specs=[
            pl.BlockSpec((_BB, _MEM), lambda i: (i, 0)),
            pl.BlockSpec((_BB, _MEM), lambda i: (i, 0)),
        ],
        out_shape=[
            jax.ShapeDtypeStruct((n_tc, _MEM), jnp.float32),
            jax.ShapeDtypeStruct((n_tc, _MEM), jnp.float32),
        ],
        scratch_shapes=[pltpu.VMEM((_MEM, _BB), jnp.float32)],
    )(query_vector, graph_out_features)

    soft_sc, logit_sc = sc_call(jnp.reshape(query_vector, (-1,)),
                                jnp.reshape(graph_out_features, (-1,)))

    soft = jnp.concatenate(
        [jnp.reshape(soft_sc, (_B_SC, _MEM)), soft_tc], axis=0)
    logits = jnp.concatenate(
        [jnp.reshape(logit_sc, (_B_SC, _MEM)), logit_tc], axis=0)
    return (soft, logits)


# speedup vs baseline: 1.0193x; 1.0172x over previous
import functools

import jax
import jax.numpy as jnp
from jax import lax
from jax.experimental import pallas as pl
from jax.experimental.pallas import tpu as pltpu
from jax.experimental.pallas import tpu_sc as plsc

_BATCH = 1024
_MEM = 200
_DIM = 128
_LANES = 16
_NW = 32
_B_SC = 512
_RPW = _B_SC // _NW
_MG = (_MEM + _LANES - 1) // _LANES
_ROW = _MEM * _DIM
_CH = _DIM // _LANES
_NBUF = 2
_BB = 64

_GATHER_DNUMS = lax.GatherDimensionNumbers(
    offset_dims=(), collapsed_slice_dims=(0,), start_index_map=(0,))


def _permute(v, idx):
    return lax.gather(v, idx[:, None], _GATHER_DNUMS, (1,),
                      mode=lax.GatherScatterMode.PROMISE_IN_BOUNDS)


def _sc_body(q_hbm, mem_hbm, soft_hbm, logit_hbm,
             q_v, mem_v, soft_v, logit_v, sem_q, sem_m0, sem_m1, sem_o):
    cid = lax.axis_index("c")
    sid = lax.axis_index("s")
    wid = sid * 2 + cid
    base = wid * _RPW

    lane = lax.iota(jnp.int32, _LANES)
    dup_mask = lane < (_MG * _LANES - _MEM)
    lane_eq = [lane == j for j in range(_LANES)]
    fold_idx = [jnp.bitwise_xor(lane, w) for w in (8, 4, 2, 1)]
    zero16 = jnp.zeros((_LANES,), jnp.float32)

    sems = (sem_m0, sem_m1)

    def start_row(b, buf):
        pltpu.async_copy(mem_hbm.at[pl.ds((base + b) * _ROW, _ROW)],
                         mem_v.at[pl.ds(buf * _ROW, _ROW)], sems[buf])

    def wait_row(b, buf):
        pltpu.make_async_copy(mem_hbm.at[pl.ds((base + b) * _ROW, _ROW)],
                              mem_v.at[pl.ds(buf * _ROW, _ROW)],
                              sems[buf]).wait()

    pltpu.async_copy(q_hbm.at[pl.ds(base * _DIM, _RPW * _DIM)], q_v, sem_q)
    for buf in range(_NBUF):
        start_row(buf, buf)
    pltpu.make_async_copy(q_hbm.at[pl.ds(base * _DIM, _RPW * _DIM)],
                          q_v, sem_q).wait()

    def crosslane_sum(v):
        for idx in fold_idx:
            v = v + _permute(v, idx)
        return v

    def dot_row(bl, buf):
        qbase = bl * _DIM
        qc = [q_v[pl.ds(qbase + k * _LANES, _LANES)] for k in range(_CH)]
        vbase = buf * _ROW

        def mg_body(mg, carry):
            col0 = jnp.minimum(mg * _LANES, _MEM - _LANES)
            goff = vbase + col0 * _DIM
            grp = zero16
            for j in range(_LANES):
                joff = goff + j * _DIM
                acc = mem_v[pl.ds(joff, _LANES)] * qc[0]
                for k in range(1, _CH):
                    acc = acc + mem_v[pl.ds(joff + k * _LANES, _LANES)] * qc[k]
                grp = jnp.where(lane_eq[j], crosslane_sum(acc), grp)
            logit_v[bl, pl.ds(col0, _LANES)] = grp
            return carry

        lax.fori_loop(0, _MG, mg_body, 0)

    _COLS = [min(mg * _LANES, _MEM - _LANES) for mg in range(_MG)]

    def softmax_row(bl):
        accs = [logit_v[bl, pl.ds(c, _LANES)] for c in _COLS]
        vmax = accs[0]
        for a in accs[1:]:
            vmax = jnp.maximum(vmax, a)
        red = vmax
        for idx in fold_idx:
            red = jnp.maximum(red, _permute(red, idx))
        exps = [jnp.exp(a - red) for a in accs]
        vsum = jnp.where(dup_mask, zero16, exps[-1])
        for e in exps[:-1]:
            vsum = vsum + e
        tot = crosslane_sum(vsum)
        inv = jnp.full((_LANES,), 1.0, jnp.float32) / tot
        for c, e in zip(_COLS, exps):
            soft_v[bl, pl.ds(c, _LANES)] = e * inv

    def rows_body(i, carry):
        for buf in range(_NBUF):
            b = _NBUF * i + buf
            wait_row(b, buf)
            dot_row(b, buf)

            @pl.when(i < _RPW // _NBUF - 1)
            def _():
                start_row(b + _NBUF, buf)

            softmax_row(b)
        return carry

    lax.fori_loop(0, _RPW // _NBUF, rows_body, 0)

    pltpu.async_copy(soft_v, soft_hbm.at[pl.ds(base, _RPW)], sem_o)
    pltpu.make_async_copy(soft_v, soft_hbm.at[pl.ds(base, _RPW)],
                          sem_o).wait()
    pltpu.sync_copy(logit_v, logit_hbm.at[pl.ds(base, _RPW)])


def _tc_body(q_ref, m_ref, soft_ref, logit_ref, lt_ref):
    qt = q_ref[...].T
    for b in range(_BB):
        lt_ref[:, b : b + 1] = jax.lax.dot(
            m_ref[b], qt[:, b : b + 1], preferred_element_type=jnp.float32
        )
    lt = lt_ref[...]
    logit_ref[...] = lt.T
    mx = jnp.max(lt, axis=0, keepdims=True)
    e = jnp.exp(lt - mx)
    soft_ref[...] = (e / jnp.sum(e, axis=0, keepdims=True)).T


@jax.jit
def kernel(query_vector, graph_out_features):
    mesh = plsc.VectorSubcoreMesh(core_axis_name="c", subcore_axis_name="s")
    sc_call = functools.partial(
        pl.kernel,
        mesh=mesh,
        compiler_params=pltpu.CompilerParams(needs_layout_passes=False,
                                             skip_device_barrier=True),
        out_type=[
            jax.ShapeDtypeStruct((_B_SC, _MEM), jnp.float32),
            jax.ShapeDtypeStruct((_B_SC, _MEM), jnp.float32),
        ],
        scratch_types=[
            pltpu.VMEM((_RPW * _DIM,), jnp.float32),
            pltpu.VMEM((_NBUF * _ROW,), jnp.float32),
            pltpu.VMEM((_RPW, _MEM), jnp.float32),
            pltpu.VMEM((_RPW, _MEM), jnp.float32),
            pltpu.SemaphoreType.DMA,
            pltpu.SemaphoreType.DMA,
            pltpu.SemaphoreType.DMA,
            pltpu.SemaphoreType.DMA,
        ],
    )(_sc_body)

    n_tc = _BATCH - _B_SC
    off = _B_SC // _BB
    soft_tc, logit_tc = pl.pallas_call(
        _tc_body,
        grid=(n_tc // _BB,),
        in_specs=[
            pl.BlockSpec((_BB, _DIM), lambda i: (i + off, 0)),
            pl.BlockSpec((_BB, _MEM, _DIM), lambda i: (i + off, 0, 0)),
        ],
        out_specs=[
            pl.BlockSpec((_BB, _MEM), lambda i: (i, 0)),
            pl.BlockSpec((_BB, _MEM), lambda i: (i, 0)),
        ],
        out_shape=[
            jax.ShapeDtypeStruct((n_tc, _MEM), jnp.float32),
            jax.ShapeDtypeStruct((n_tc, _MEM), jnp.float32),
        ],
        scratch_shapes=[pltpu.VMEM((_MEM, _BB), jnp.float32)],
    )(query_vector, graph_out_features)

    soft_sc, logit_sc = sc_call(jnp.reshape(query_vector, (-1,)),
                                jnp.reshape(graph_out_features, (-1,)))

    soft = jnp.concatenate([soft_sc, soft_tc], axis=0)
    logits = jnp.concatenate([logit_sc, logit_tc], axis=0)
    return (soft, logits)


# speedup vs baseline: 1.0344x; 1.0148x over previous
import functools

import jax
import jax.numpy as jnp
from jax import lax
from jax.experimental import pallas as pl
from jax.experimental.pallas import tpu as pltpu
from jax.experimental.pallas import tpu_sc as plsc

_BATCH = 1024
_MEM = 200
_DIM = 128
_LANES = 16
_NW = 32
_B_SC = 256
_RPW = _B_SC // _NW
_MG = (_MEM + _LANES - 1) // _LANES
_ROW = _MEM * _DIM
_CH = _DIM // _LANES
_NBUF = 2
_BB = 64

_GATHER_DNUMS = lax.GatherDimensionNumbers(
    offset_dims=(), collapsed_slice_dims=(0,), start_index_map=(0,))


def _permute(v, idx):
    return lax.gather(v, idx[:, None], _GATHER_DNUMS, (1,),
                      mode=lax.GatherScatterMode.PROMISE_IN_BOUNDS)


def _sc_body(q_hbm, mem_hbm, soft_hbm, logit_hbm,
             q_v, mem_v, soft_v, logit_v, sem_q, sem_m0, sem_m1, sem_o):
    cid = lax.axis_index("c")
    sid = lax.axis_index("s")
    wid = sid * 2 + cid
    base = wid * _RPW

    lane = lax.iota(jnp.int32, _LANES)
    dup_mask = lane < (_MG * _LANES - _MEM)
    lane_eq = [lane == j for j in range(_LANES)]
    fold_idx = [jnp.bitwise_xor(lane, w) for w in (8, 4, 2, 1)]
    zero16 = jnp.zeros((_LANES,), jnp.float32)

    sems = (sem_m0, sem_m1)

    def start_row(b, buf):
        pltpu.async_copy(mem_hbm.at[pl.ds((base + b) * _ROW, _ROW)],
                         mem_v.at[pl.ds(buf * _ROW, _ROW)], sems[buf])

    def wait_row(b, buf):
        pltpu.make_async_copy(mem_hbm.at[pl.ds((base + b) * _ROW, _ROW)],
                              mem_v.at[pl.ds(buf * _ROW, _ROW)],
                              sems[buf]).wait()

    pltpu.async_copy(q_hbm.at[pl.ds(base * _DIM, _RPW * _DIM)], q_v, sem_q)
    for buf in range(_NBUF):
        start_row(buf, buf)
    pltpu.make_async_copy(q_hbm.at[pl.ds(base * _DIM, _RPW * _DIM)],
                          q_v, sem_q).wait()

    def crosslane_sum(v):
        for idx in fold_idx:
            v = v + _permute(v, idx)
        return v

    def dot_row(bl, buf):
        qbase = bl * _DIM
        qc = [q_v[pl.ds(qbase + k * _LANES, _LANES)] for k in range(_CH)]
        vbase = buf * _ROW

        def mg_body(mg, carry):
            col0 = jnp.minimum(mg * _LANES, _MEM - _LANES)
            goff = vbase + col0 * _DIM
            grp = zero16
            for j in range(_LANES):
                joff = goff + j * _DIM
                acc = mem_v[pl.ds(joff, _LANES)] * qc[0]
                for k in range(1, _CH):
                    acc = acc + mem_v[pl.ds(joff + k * _LANES, _LANES)] * qc[k]
                grp = jnp.where(lane_eq[j], crosslane_sum(acc), grp)
            logit_v[bl, pl.ds(col0, _LANES)] = grp
            return carry

        lax.fori_loop(0, _MG, mg_body, 0)

    _COLS = [min(mg * _LANES, _MEM - _LANES) for mg in range(_MG)]

    def softmax_row(bl):
        accs = [logit_v[bl, pl.ds(c, _LANES)] for c in _COLS]
        vmax = accs[0]
        for a in accs[1:]:
            vmax = jnp.maximum(vmax, a)
        red = vmax
        for idx in fold_idx:
            red = jnp.maximum(red, _permute(red, idx))
        exps = [jnp.exp(a - red) for a in accs]
        vsum = jnp.where(dup_mask, zero16, exps[-1])
        for e in exps[:-1]:
            vsum = vsum + e
        tot = crosslane_sum(vsum)
        inv = jnp.full((_LANES,), 1.0, jnp.float32) / tot
        for c, e in zip(_COLS, exps):
            soft_v[bl, pl.ds(c, _LANES)] = e * inv

    def rows_body(i, carry):
        for buf in range(_NBUF):
            b = _NBUF * i + buf
            wait_row(b, buf)
            dot_row(b, buf)

            @pl.when(i < _RPW // _NBUF - 1)
            def _():
                start_row(b + _NBUF, buf)

            softmax_row(b)
        return carry

    lax.fori_loop(0, _RPW // _NBUF, rows_body, 0)

    pltpu.async_copy(soft_v, soft_hbm.at[pl.ds(base, _RPW)], sem_o)
    pltpu.make_async_copy(soft_v, soft_hbm.at[pl.ds(base, _RPW)],
                          sem_o).wait()
    pltpu.sync_copy(logit_v, logit_hbm.at[pl.ds(base, _RPW)])


def _tc_body(q_ref, m_ref, soft_ref, logit_ref, lt_ref):
    qt = q_ref[...].T
    for b in range(_BB):
        lt_ref[:, b : b + 1] = jax.lax.dot(
            m_ref[b], qt[:, b : b + 1], preferred_element_type=jnp.float32
        )
    lt = lt_ref[...]
    logit_ref[...] = lt.T
    mx = jnp.max(lt, axis=0, keepdims=True)
    e = jnp.exp(lt - mx)
    soft_ref[...] = (e / jnp.sum(e, axis=0, keepdims=True)).T


@jax.jit
def kernel(query_vector, graph_out_features):
    mesh = plsc.VectorSubcoreMesh(core_axis_name="c", subcore_axis_name="s")
    sc_call = functools.partial(
        pl.kernel,
        mesh=mesh,
        compiler_params=pltpu.CompilerParams(needs_layout_passes=False,
                                             skip_device_barrier=True),
        out_type=[
            jax.ShapeDtypeStruct((_B_SC, _MEM), jnp.float32),
            jax.ShapeDtypeStruct((_B_SC, _MEM), jnp.float32),
        ],
        scratch_types=[
            pltpu.VMEM((_RPW * _DIM,), jnp.float32),
            pltpu.VMEM((_NBUF * _ROW,), jnp.float32),
            pltpu.VMEM((_RPW, _MEM), jnp.float32),
            pltpu.VMEM((_RPW, _MEM), jnp.float32),
            pltpu.SemaphoreType.DMA,
            pltpu.SemaphoreType.DMA,
            pltpu.SemaphoreType.DMA,
            pltpu.SemaphoreType.DMA,
        ],
    )(_sc_body)

    n_tc = _BATCH - _B_SC
    off = _B_SC // _BB
    soft_tc, logit_tc = pl.pallas_call(
        _tc_body,
        grid=(n_tc // _BB,),
        in_specs=[
            pl.BlockSpec((_BB, _DIM), lambda i: (i + off, 0)),
            pl.BlockSpec((_BB, _MEM, _DIM), lambda i: (i + off, 0, 0)),
        ],
        out_specs=[
            pl.BlockSpec((_BB, _MEM), lambda i: (i, 0)),
            pl.BlockSpec((_BB, _MEM), lambda i: (i, 0)),
        ],
        out_shape=[
            jax.ShapeDtypeStruct((n_tc, _MEM), jnp.float32),
            jax.ShapeDtypeStruct((n_tc, _MEM), jnp.float32),
        ],
        scratch_shapes=[pltpu.VMEM((_MEM, _BB), jnp.float32)],
    )(query_vector, graph_out_features)

    soft_sc, logit_sc = sc_call(jnp.reshape(query_vector, (-1,)),
                                jnp.reshape(graph_out_features, (-1,)))

    soft = jnp.concatenate([soft_sc, soft_tc], axis=0)
    logits = jnp.concatenate([logit_sc, logit_tc], axis=0)
    return (soft, logits)
